# baseline (speedup 1.0000x reference)
import functools

import jax
import jax.numpy as jnp
from jax import lax
from jax.experimental import pallas as pl
from jax.experimental.pallas import tpu as pltpu
from jax.experimental.pallas import tpu_sc as plsc

_N = 10000
_E = 320000
_D = 128
_H = 64
_G = 64
_C = 10

_NSC = 2
_NTILE = 16
_NW = _NSC * _NTILE
_K = 125
_NCH = 80
_GSZ = 4
_NGRP = _NCH // _GSZ
_NP = 10240
_RPT = _NP // _NTILE

_NB = 5
_BN = _N // _NB
_BF = _BN // 2
_NF = _N // 2


@functools.partial(
    pl.kernel,
    out_type=jax.ShapeDtypeStruct((_NSC, _NP, _H), jnp.float32),
    mesh=plsc.VectorSubcoreMesh(core_axis_name="c", subcore_axis_name="s"),
    scratch_types=[
        pltpu.VMEM((_NCH, _K), jnp.int32),
        pltpu.VMEM((_NCH, _K), jnp.int32),
        pltpu.VMEM((2, _GSZ, _K, _H), jnp.float32),
        pltpu.VMEM_SHARED((_NP, _H), jnp.float32),
        pltpu.SemaphoreType.DMA,
        pltpu.SemaphoreType.DMA,
        pltpu.SemaphoreType.DMA,
        pltpu.SemaphoreType.DMA,
    ],
    compiler_params=pltpu.CompilerParams(use_tc_tiling_on_sc=False),
)
def _sc_agg(p_hbm, src_hbm, dst_hbm, zero_hbm, out_hbm, srcv, dstv, rows, acc,
            sga, sgb, ssa, ssb):
    c = lax.axis_index("c")
    s = lax.axis_index("s")
    wid = c * _NTILE + s
    pltpu.sync_copy(zero_hbm.at[pl.ds(s * _RPT, _RPT)], acc.at[pl.ds(s * _RPT, _RPT)])
    pltpu.sync_copy(src_hbm.at[wid], srcv)
    pltpu.sync_copy(dst_hbm.at[wid], dstv)
    plsc.subcore_barrier()

    def fire(bank, g, sem):
        for t in range(_GSZ):
            pltpu.async_copy(p_hbm.at[srcv.at[g * _GSZ + t]],
                             rows.at[bank, t], sem)

    def drain(bank, g, semg, sems):
        for t in range(_GSZ):
            pltpu.make_async_copy(p_hbm.at[srcv.at[g * _GSZ + t]],
                                  rows.at[bank, t], semg).wait()
        for t in range(_GSZ):
            pltpu.async_copy(rows.at[bank, t],
                             acc.at[dstv.at[g * _GSZ + t]], sems, add=True)
        for t in range(_GSZ):
            pltpu.make_async_copy(rows.at[bank, t],
                                  acc.at[dstv.at[g * _GSZ + t]], sems).wait()

    fire(0, 0, sga)

    def body(i, carry):
        fire(1, 2 * i + 1, sgb)
        drain(0, 2 * i, sga, ssa)

        @pl.when(i < _NGRP // 2 - 1)
        def _next():
            fire(0, 2 * i + 2, sga)

        drain(1, 2 * i + 1, sgb, ssb)
        return carry

    lax.fori_loop(0, _NGRP // 2, body, 0)
    plsc.subcore_barrier()
    pltpu.sync_copy(acc.at[pl.ds(s * _RPT, _RPT)],
                    out_hbm.at[c, pl.ds(s * _RPT, _RPT)])


def _proj_body(xa_ref, xb_ref, w_ref, o_ref):
    w = w_ref[...]
    a = jnp.dot(xa_ref[...], w, preferred_element_type=jnp.float32)
    b = jnp.dot(xb_ref[...], w, preferred_element_type=jnp.float32)
    o_ref[...] = jnp.concatenate([a, b], axis=1)


def _proj(x, w):
    return pl.pallas_call(
        _proj_body,
        grid=(_NB,),
        in_specs=[
            pl.BlockSpec((_BF, _D), lambda i: (i, 0)),
            pl.BlockSpec((_BF, _D), lambda i: (i + _NB, 0)),
            pl.BlockSpec((_D, _H), lambda i: (0, 0)),
        ],
        out_specs=pl.BlockSpec((_BF, 2 * _H), lambda i: (i, 0)),
        out_shape=jax.ShapeDtypeStruct((_NF, 2 * _H), jnp.float32),
    )(x, x, w)


def _mlp_body(has_next, p_ref, agg_ref, b1_ref, w2_ref, b2_ref, w1n_ref,
              bt_ref, bb_ref, *refs):
    if has_next:
        pn_ref, s_out, m_out, c_out = refs
    else:
        s_out, m_out, c_out = refs
    i = pl.program_id(0)

    @pl.when(i == 0)
    def _init():
        s_out[...] = jnp.zeros_like(s_out)
        m_out[...] = jnp.full_like(m_out, -jnp.inf)
        c_out[...] = jnp.zeros_like(c_out)

    m = 2.0 * p_ref[...] + agg_ref[0] + agg_ref[1] + b1_ref[...]
    m = jnp.maximum(m, 0.0)
    w2 = w2_ref[...]
    b2 = b2_ref[...]
    h_top = jnp.maximum(
        jnp.dot(m[:, :_H], w2, preferred_element_type=jnp.float32) + b2, 0.0)
    h_bot = jnp.maximum(
        jnp.dot(m[:, _H:], w2, preferred_element_type=jnp.float32) + b2, 0.0)
    if has_next:
        w1n = w1n_ref[...]
        pn_ref[...] = jnp.concatenate(
            [jnp.dot(h_top, w1n, preferred_element_type=jnp.float32),
             jnp.dot(h_bot, w1n, preferred_element_type=jnp.float32)], axis=1)

    bt = bt_ref[...]
    bb = bb_ref[...]
    iota = lax.broadcasted_iota(jnp.int32, (_BF, _G), 1)
    oh_t = (bt == iota).astype(jnp.float32)
    oh_b = (bb == iota).astype(jnp.float32)
    dn = (((0,), (0,)), ((), ()))
    s_out[...] += (lax.dot_general(oh_t, h_top, dn, preferred_element_type=jnp.float32)
                   + lax.dot_general(oh_b, h_bot, dn, preferred_element_type=jnp.float32))
    ones = jnp.ones((_BF, 8), jnp.float32)
    c_out[...] += (lax.dot_general(oh_t, ones, dn, preferred_element_type=jnp.float32)
                   + lax.dot_general(oh_b, ones, dn, preferred_element_type=jnp.float32))

    rowid = lax.broadcasted_iota(jnp.int32, (_G, 1), 0)

    def g_top(g, carry):
        mg = jnp.max(jnp.where(bt == g, h_top, -jnp.inf), axis=0, keepdims=True)
        m_out[...] = jnp.maximum(m_out[...], jnp.where(rowid == g, mg, -jnp.inf))
        return carry

    def g_bot(g, carry):
        mg = jnp.max(jnp.where(bb == g, h_bot, -jnp.inf), axis=0, keepdims=True)
        m_out[...] = jnp.maximum(m_out[...], jnp.where(rowid == g, mg, -jnp.inf))
        return carry

    lax.fori_loop(bt[0, 0], bt[_BF - 1, 0] + 1, g_top, 0)
    lax.fori_loop(bb[0, 0], bb[_BF - 1, 0] + 1, g_bot, 0)


def _mlp(p, agg, b1f, w2, b2, w1n, bat, has_next):
    in_specs = [
        pl.BlockSpec((_BF, 2 * _H), lambda i: (i, 0)),
        pl.BlockSpec((_NSC, _BF, 2 * _H), lambda i: (0, i, 0)),
        pl.BlockSpec((1, 2 * _H), lambda i: (0, 0)),
        pl.BlockSpec((_H, _H), lambda i: (0, 0)),
        pl.BlockSpec((1, _H), lambda i: (0, 0)),
        pl.BlockSpec((_H, _H), lambda i: (0, 0)),
        pl.BlockSpec((_BF, 1), lambda i: (i, 0)),
        pl.BlockSpec((_BF, 1), lambda i: (i + _NB, 0)),
    ]
    pool_specs = [pl.BlockSpec((_G, _H), lambda i: (0, 0)),
                  pl.BlockSpec((_G, _H), lambda i: (0, 0)),
                  pl.BlockSpec((_G, 8), lambda i: (0, 0))]
    pool_shapes = [jax.ShapeDtypeStruct((_G, _H), jnp.float32),
                   jax.ShapeDtypeStruct((_G, _H), jnp.float32),
                   jax.ShapeDtypeStruct((_G, 8), jnp.float32)]
    if has_next:
        out_specs = [pl.BlockSpec((_BF, 2 * _H), lambda i: (i, 0))] + pool_specs
        out_shape = [jax.ShapeDtypeStruct((_NF, 2 * _H), jnp.float32)] + pool_shapes
    else:
        out_specs = pool_specs
        out_shape = pool_shapes
    return pl.pallas_call(
        functools.partial(_mlp_body, has_next),
        grid=(_NB,),
        in_specs=in_specs,
        out_specs=out_specs,
        out_shape=out_shape,
    )(p, agg, b1f, w2, b2, w1n, bat, bat)


def _head_body(s1, s2, s3, m1, m2, m3, cnt_ref, fc1w_ref, fc1b_ref,
               fc2w_ref, fc2b_ref, out_ref):
    cnt = cnt_ref[:, 0:1]
    inv = 1.0 / jnp.maximum(cnt, 1.0)
    w = fc1w_ref[...]
    z = fc1b_ref[...]
    sums = [s1[...], s2[...], s3[...]]
    for k in range(3):
        mean_k = sums[k] * inv
        z = z + jnp.dot(mean_k, w[64 * k:64 * (k + 1)],
                        preferred_element_type=jnp.float32)
    maxs = [m1[...], m2[...], m3[...]]
    for k in range(3):
        mx_k = jnp.where(cnt > 0.0, maxs[k], 0.0)
        z = z + jnp.dot(mx_k, w[192 + 64 * k:192 + 64 * (k + 1)],
                        preferred_element_type=jnp.float32)
    for k in range(3):
        z = z + jnp.dot(sums[k], w[384 + 64 * k:384 + 64 * (k + 1)],
                        preferred_element_type=jnp.float32)
    z = jnp.maximum(z, 0.0)
    o = jnp.dot(z, fc2w_ref[...], preferred_element_type=jnp.float32) \
        + fc2b_ref[...]
    out_ref[...] = 1.0 / (1.0 + jnp.exp(-o))


def _head(pools, fc1_w, fc1_b, fc2_w, fc2_b):
    (s1, m1, c1), (s2, m2, _), (s3, m3, _) = pools
    gspec = lambda shape: pl.BlockSpec(shape, lambda: tuple(0 for _ in shape))
    return pl.pallas_call(
        _head_body,
        in_specs=[gspec((_G, _H))] * 6 + [
            gspec((_G, 8)), gspec((9 * _H, _H)), gspec((1, _H)),
            gspec((_H, _C)), gspec((1, _C)),
        ],
        out_specs=gspec((_G, _C)),
        out_shape=jax.ShapeDtypeStruct((_G, _C), jnp.float32),
    )(s1, s2, s3, m1, m2, m3, c1, fc1_w, fc1_b, fc2_w, fc2_b)


def _remap_fold(idx):
    return jnp.where(idx < _NF, 2 * idx, 2 * idx - (_N - 1))


def kernel(x, edge_index, batch,
           c0_w1, c0_b1, c0_w2, c0_b2,
           c1_w1, c1_b1, c1_w2, c1_b2,
           c2_w1, c2_b1, c2_w2, c2_b2,
           fc1_w, fc1_b, fc2_w, fc2_b):
    src = _remap_fold(edge_index[0].astype(jnp.int32)).reshape(_NW, _NCH, _K)
    dst = _remap_fold(edge_index[1].astype(jnp.int32)).reshape(_NW, _NCH, _K)
    zeros = jnp.zeros((_NP, _H), jnp.float32)
    bat = batch.astype(jnp.int32).reshape(_N, 1)

    params = [(c0_b1, c0_w2, c0_b2), (c1_b1, c1_w2, c1_b2), (c2_b1, c2_w2, c2_b2)]
    next_w1 = [c1_w1, c2_w1, None]

    p = _proj(x, c0_w1)
    pools = []
    for l in range(3):
        agg = _sc_agg(p.reshape(_N, _H), src, dst, zeros)
        agg_f = agg.reshape(_NSC, _NP // 2, 2 * _H)
        b1, w2, b2 = params[l]
        b1f = jnp.concatenate([b1, b1]).reshape(1, 2 * _H)
        has_next = next_w1[l] is not None
        w1n = next_w1[l] if has_next else w2
        res = _mlp(p, agg_f, b1f, w2, b2.reshape(1, _H), w1n, bat, has_next)
        if has_next:
            p = res[0]
            pools.append(res[1:])
        else:
            pools.append(res)

    return _head(pools, fc1_w, fc1_b.reshape(1, _H), fc2_w,
                 fc2_b.reshape(1, _C))


# speedup vs baseline: 1.0635x; 1.0635x over previous
import functools

import jax
import jax.numpy as jnp
from jax import lax
from jax.experimental import pallas as pl
from jax.experimental.pallas import tpu as pltpu
from jax.experimental.pallas import tpu_sc as plsc

_N = 10000
_E = 320000
_D = 128
_H = 64
_G = 64
_C = 10

_NSC = 2
_NTILE = 16
_NW = _NSC * _NTILE
_K = 125
_NCH = 80
_GSZ = 4
_NGRP = _NCH // _GSZ
_NP = 10240
_RPT = _NP // _NTILE

_NB = 5
_BN = _N // _NB
_BF = _BN // 2
_NF = _N // 2


@functools.partial(
    pl.kernel,
    out_type=jax.ShapeDtypeStruct((_NSC, _NP, _H), jnp.float32),
    mesh=plsc.VectorSubcoreMesh(core_axis_name="c", subcore_axis_name="s"),
    scratch_types=[
        pltpu.VMEM((_NCH, _K), jnp.int32),
        pltpu.VMEM((_NCH, _K), jnp.int32),
        pltpu.VMEM((2, _GSZ, _K, _H), jnp.float32),
        pltpu.VMEM_SHARED((_NP, _H), jnp.float32),
        pltpu.SemaphoreType.DMA,
        pltpu.SemaphoreType.DMA,
        pltpu.SemaphoreType.DMA,
        pltpu.SemaphoreType.DMA,
    ],
    compiler_params=pltpu.CompilerParams(use_tc_tiling_on_sc=False),
)
def _sc_agg(p_hbm, src_hbm, dst_hbm, zero_hbm, out_hbm, srcv, dstv, rows, acc,
            sga, sgb, ssa, ssb):
    c = lax.axis_index("c")
    s = lax.axis_index("s")
    wid = c * _NTILE + s
    pltpu.sync_copy(zero_hbm.at[pl.ds(s * _RPT, _RPT)], acc.at[pl.ds(s * _RPT, _RPT)])
    pltpu.sync_copy(src_hbm.at[wid], srcv)
    pltpu.sync_copy(dst_hbm.at[wid], dstv)
    plsc.subcore_barrier()

    def fire(bank, g, sem):
        for t in range(_GSZ):
            pltpu.async_copy(p_hbm.at[srcv.at[g * _GSZ + t]],
                             rows.at[bank, t], sem)

    def drain(bank, g, semg, sems):
        for t in range(_GSZ):
            pltpu.make_async_copy(p_hbm.at[srcv.at[g * _GSZ + t]],
                                  rows.at[bank, t], semg).wait()
        for t in range(_GSZ):
            pltpu.async_copy(rows.at[bank, t],
                             acc.at[dstv.at[g * _GSZ + t]], sems, add=True)
        for t in range(_GSZ):
            pltpu.make_async_copy(rows.at[bank, t],
                                  acc.at[dstv.at[g * _GSZ + t]], sems).wait()

    fire(0, 0, sga)

    def body(i, carry):
        fire(1, 2 * i + 1, sgb)
        drain(0, 2 * i, sga, ssa)

        @pl.when(i < _NGRP // 2 - 1)
        def _next():
            fire(0, 2 * i + 2, sga)

        drain(1, 2 * i + 1, sgb, ssb)
        return carry

    lax.fori_loop(0, _NGRP // 2, body, 0)
    plsc.subcore_barrier()
    pltpu.sync_copy(acc.at[pl.ds(s * _RPT, _RPT)],
                    out_hbm.at[c, pl.ds(s * _RPT, _RPT)])


def _proj_body(xa_ref, xb_ref, w_ref, o_ref):
    w = w_ref[...]
    a = jnp.dot(xa_ref[...], w, preferred_element_type=jnp.float32)
    b = jnp.dot(xb_ref[...], w, preferred_element_type=jnp.float32)
    o_ref[...] = jnp.concatenate([a, b], axis=1)


def _proj(x, w):
    return pl.pallas_call(
        _proj_body,
        grid=(_NB,),
        in_specs=[
            pl.BlockSpec((_BF, _D), lambda i: (i, 0)),
            pl.BlockSpec((_BF, _D), lambda i: (i + _NB, 0)),
            pl.BlockSpec((_D, _H), lambda i: (0, 0)),
        ],
        out_specs=pl.BlockSpec((_BF, 2 * _H), lambda i: (i, 0)),
        out_shape=jax.ShapeDtypeStruct((_NF, 2 * _H), jnp.float32),
    )(x, x, w)


def _mlp_body(has_next, p_ref, agg_ref, b1_ref, w2_ref, b2_ref, w1n_ref,
              bt_ref, bb_ref, *refs):
    if has_next:
        pn_ref, h_out, s_out, c_out = refs
    else:
        h_out, s_out, c_out = refs
    i = pl.program_id(0)

    @pl.when(i == 0)
    def _init():
        s_out[...] = jnp.zeros_like(s_out)
        c_out[...] = jnp.zeros_like(c_out)

    m = 2.0 * p_ref[...] + agg_ref[0] + agg_ref[1] + b1_ref[...]
    m = jnp.maximum(m, 0.0)
    w2 = w2_ref[...]
    b2 = b2_ref[...]
    h_top = jnp.maximum(
        jnp.dot(m[:, :_H], w2, preferred_element_type=jnp.float32) + b2, 0.0)
    h_bot = jnp.maximum(
        jnp.dot(m[:, _H:], w2, preferred_element_type=jnp.float32) + b2, 0.0)
    h_out[...] = jnp.concatenate([h_top, h_bot], axis=1)
    if has_next:
        w1n = w1n_ref[...]
        pn_ref[...] = jnp.concatenate(
            [jnp.dot(h_top, w1n, preferred_element_type=jnp.float32),
             jnp.dot(h_bot, w1n, preferred_element_type=jnp.float32)], axis=1)

    bt = bt_ref[...]
    bb = bb_ref[...]
    iota = lax.broadcasted_iota(jnp.int32, (_BF, _G), 1)
    oh_t = (bt == iota).astype(jnp.float32)
    oh_b = (bb == iota).astype(jnp.float32)
    dn = (((0,), (0,)), ((), ()))
    s_out[...] += (lax.dot_general(oh_t, h_top, dn, preferred_element_type=jnp.float32)
                   + lax.dot_general(oh_b, h_bot, dn, preferred_element_type=jnp.float32))
    ones = jnp.ones((_BF, 8), jnp.float32)
    c_out[...] += (lax.dot_general(oh_t, ones, dn, preferred_element_type=jnp.float32)
                   + lax.dot_general(oh_b, ones, dn, preferred_element_type=jnp.float32))

def _maxpool_body(h_ref, bt_ref, bb_ref, m_out):
    i = pl.program_id(0)

    @pl.when(i == 0)
    def _init():
        m_out[...] = jnp.full_like(m_out, -jnp.inf)

    h = h_ref[...]
    h_top = h[:, :_H]
    h_bot = h[:, _H:]
    bt = bt_ref[...]
    bb = bb_ref[...]
    rowid = lax.broadcasted_iota(jnp.int32, (_G, 1), 0)

    def g_top(g, carry):
        mg = jnp.max(jnp.where(bt == g, h_top, -jnp.inf), axis=0, keepdims=True)
        m_out[...] = jnp.maximum(m_out[...], jnp.where(rowid == g, mg, -jnp.inf))
        return carry

    def g_bot(g, carry):
        mg = jnp.max(jnp.where(bb == g, h_bot, -jnp.inf), axis=0, keepdims=True)
        m_out[...] = jnp.maximum(m_out[...], jnp.where(rowid == g, mg, -jnp.inf))
        return carry

    lax.fori_loop(bt[0, 0], bt[_BF - 1, 0] + 1, g_top, 0)
    lax.fori_loop(bb[0, 0], bb[_BF - 1, 0] + 1, g_bot, 0)


def _maxpool(h, bat):
    return pl.pallas_call(
        _maxpool_body,
        grid=(_NB,),
        in_specs=[
            pl.BlockSpec((_BF, 2 * _H), lambda i: (i, 0)),
            pl.BlockSpec((_BF, 1), lambda i: (i, 0)),
            pl.BlockSpec((_BF, 1), lambda i: (i + _NB, 0)),
        ],
        out_specs=pl.BlockSpec((_G, _H), lambda i: (0, 0)),
        out_shape=jax.ShapeDtypeStruct((_G, _H), jnp.float32),
    )(h, bat, bat)


def _mlp(p, agg, b1f, w2, b2, w1n, bat, has_next):
    in_specs = [
        pl.BlockSpec((_BF, 2 * _H), lambda i: (i, 0)),
        pl.BlockSpec((_NSC, _BF, 2 * _H), lambda i: (0, i, 0)),
        pl.BlockSpec((1, 2 * _H), lambda i: (0, 0)),
        pl.BlockSpec((_H, _H), lambda i: (0, 0)),
        pl.BlockSpec((1, _H), lambda i: (0, 0)),
        pl.BlockSpec((_H, _H), lambda i: (0, 0)),
        pl.BlockSpec((_BF, 1), lambda i: (i, 0)),
        pl.BlockSpec((_BF, 1), lambda i: (i + _NB, 0)),
    ]
    pool_specs = [pl.BlockSpec((_BF, 2 * _H), lambda i: (i, 0)),
                  pl.BlockSpec((_G, _H), lambda i: (0, 0)),
                  pl.BlockSpec((_G, 8), lambda i: (0, 0))]
    pool_shapes = [jax.ShapeDtypeStruct((_NF, 2 * _H), jnp.float32),
                   jax.ShapeDtypeStruct((_G, _H), jnp.float32),
                   jax.ShapeDtypeStruct((_G, 8), jnp.float32)]
    if has_next:
        out_specs = [pl.BlockSpec((_BF, 2 * _H), lambda i: (i, 0))] + pool_specs
        out_shape = [jax.ShapeDtypeStruct((_NF, 2 * _H), jnp.float32)] + pool_shapes
    else:
        out_specs = pool_specs
        out_shape = pool_shapes
    return pl.pallas_call(
        functools.partial(_mlp_body, has_next),
        grid=(_NB,),
        in_specs=in_specs,
        out_specs=out_specs,
        out_shape=out_shape,
    )(p, agg, b1f, w2, b2, w1n, bat, bat)


def _head_body(s1, s2, s3, m1, m2, m3, cnt_ref, fc1w_ref, fc1b_ref,
               fc2w_ref, fc2b_ref, out_ref):
    cnt = cnt_ref[:, 0:1]
    inv = 1.0 / jnp.maximum(cnt, 1.0)
    w = fc1w_ref[...]
    z = fc1b_ref[...]
    sums = [s1[...], s2[...], s3[...]]
    for k in range(3):
        mean_k = sums[k] * inv
        z = z + jnp.dot(mean_k, w[64 * k:64 * (k + 1)],
                        preferred_element_type=jnp.float32)
    maxs = [m1[...], m2[...], m3[...]]
    for k in range(3):
        mx_k = jnp.where(cnt > 0.0, maxs[k], 0.0)
        z = z + jnp.dot(mx_k, w[192 + 64 * k:192 + 64 * (k + 1)],
                        preferred_element_type=jnp.float32)
    for k in range(3):
        z = z + jnp.dot(sums[k], w[384 + 64 * k:384 + 64 * (k + 1)],
                        preferred_element_type=jnp.float32)
    z = jnp.maximum(z, 0.0)
    o = jnp.dot(z, fc2w_ref[...], preferred_element_type=jnp.float32) \
        + fc2b_ref[...]
    out_ref[...] = 1.0 / (1.0 + jnp.exp(-o))


def _head(pools, fc1_w, fc1_b, fc2_w, fc2_b):
    (s1, m1, c1), (s2, m2, _), (s3, m3, _) = pools
    gspec = lambda shape: pl.BlockSpec(shape, lambda: tuple(0 for _ in shape))
    return pl.pallas_call(
        _head_body,
        in_specs=[gspec((_G, _H))] * 6 + [
            gspec((_G, 8)), gspec((9 * _H, _H)), gspec((1, _H)),
            gspec((_H, _C)), gspec((1, _C)),
        ],
        out_specs=gspec((_G, _C)),
        out_shape=jax.ShapeDtypeStruct((_G, _C), jnp.float32),
    )(s1, s2, s3, m1, m2, m3, c1, fc1_w, fc1_b, fc2_w, fc2_b)


def _remap_fold(idx):
    return jnp.where(idx < _NF, 2 * idx, 2 * idx - (_N - 1))


def kernel(x, edge_index, batch,
           c0_w1, c0_b1, c0_w2, c0_b2,
           c1_w1, c1_b1, c1_w2, c1_b2,
           c2_w1, c2_b1, c2_w2, c2_b2,
           fc1_w, fc1_b, fc2_w, fc2_b):
    src = _remap_fold(edge_index[0].astype(jnp.int32)).reshape(_NW, _NCH, _K)
    dst = _remap_fold(edge_index[1].astype(jnp.int32)).reshape(_NW, _NCH, _K)
    zeros = jnp.zeros((_NP, _H), jnp.float32)
    bat = batch.astype(jnp.int32).reshape(_N, 1)

    params = [(c0_b1, c0_w2, c0_b2), (c1_b1, c1_w2, c1_b2), (c2_b1, c2_w2, c2_b2)]
    next_w1 = [c1_w1, c2_w1, None]

    p = _proj(x, c0_w1)
    pools = []
    for l in range(3):
        agg = _sc_agg(p.reshape(_N, _H), src, dst, zeros)
        agg_f = agg.reshape(_NSC, _NP // 2, 2 * _H)
        b1, w2, b2 = params[l]
        b1f = jnp.concatenate([b1, b1]).reshape(1, 2 * _H)
        has_next = next_w1[l] is not None
        w1n = next_w1[l] if has_next else w2
        res = _mlp(p, agg_f, b1f, w2, b2.reshape(1, _H), w1n, bat, has_next)
        if has_next:
            p, h, s, c = res
        else:
            h, s, c = res
        pools.append((s, _maxpool(h, bat), c))

    return _head(pools, fc1_w, fc1_b.reshape(1, _H), fc2_w,
                 fc2_b.reshape(1, _C))


# speedup vs baseline: 1.1890x; 1.1180x over previous
import functools

import jax
import jax.numpy as jnp
from jax import lax
from jax.experimental import pallas as pl
from jax.experimental.pallas import tpu as pltpu
from jax.experimental.pallas import tpu_sc as plsc

_N = 10000
_E = 320000
_D = 128
_H = 64
_G = 64
_C = 10

_NSC = 2
_NTILE = 16
_NW = _NSC * _NTILE
_K = 125
_NCH = 80
_GSZ = 4
_NGRP = _NCH // _GSZ
_NP = 10240
_RPT = _NP // _NTILE

_NB = 5
_BN = _N // _NB
_BF = _BN // 2
_NF = _N // 2


@functools.partial(
    pl.kernel,
    out_type=jax.ShapeDtypeStruct((_NSC, _NP, _H), jnp.float32),
    mesh=plsc.VectorSubcoreMesh(core_axis_name="c", subcore_axis_name="s"),
    scratch_types=[
        pltpu.VMEM((_NCH, _K), jnp.int32),
        pltpu.VMEM((_NCH, _K), jnp.int32),
        pltpu.VMEM((2, _GSZ, _K, _H), jnp.float32),
        pltpu.VMEM_SHARED((_NP, _H), jnp.float32),
        pltpu.SemaphoreType.DMA,
        pltpu.SemaphoreType.DMA,
        pltpu.SemaphoreType.DMA,
        pltpu.SemaphoreType.DMA,
    ],
    compiler_params=pltpu.CompilerParams(use_tc_tiling_on_sc=False),
)
def _sc_agg(p_hbm, ed_hbm, zero_hbm, out_hbm, srcv, dstv, rows, acc,
            sga, sgb, ssa, ssb):
    c = lax.axis_index("c")
    s = lax.axis_index("s")
    wid = c * _NTILE + s
    pltpu.sync_copy(zero_hbm.at[pl.ds(s * _RPT, _RPT)], acc.at[pl.ds(s * _RPT, _RPT)])
    pltpu.sync_copy(ed_hbm.at[0, wid], srcv)
    pltpu.sync_copy(ed_hbm.at[1, wid], dstv)
    plsc.subcore_barrier()

    def fire(bank, g, sem):
        for t in range(_GSZ):
            pltpu.async_copy(p_hbm.at[srcv.at[g * _GSZ + t]],
                             rows.at[bank, t], sem)

    def drain(bank, g, semg, sems):
        for t in range(_GSZ):
            pltpu.make_async_copy(p_hbm.at[srcv.at[g * _GSZ + t]],
                                  rows.at[bank, t], semg).wait()
        for t in range(_GSZ):
            pltpu.async_copy(rows.at[bank, t],
                             acc.at[dstv.at[g * _GSZ + t]], sems, add=True)
        for t in range(_GSZ):
            pltpu.make_async_copy(rows.at[bank, t],
                                  acc.at[dstv.at[g * _GSZ + t]], sems).wait()

    fire(0, 0, sga)

    def body(i, carry):
        fire(1, 2 * i + 1, sgb)
        drain(0, 2 * i, sga, ssa)

        @pl.when(i < _NGRP // 2 - 1)
        def _next():
            fire(0, 2 * i + 2, sga)

        drain(1, 2 * i + 1, sgb, ssb)
        return carry

    lax.fori_loop(0, _NGRP // 2, body, 0)
    plsc.subcore_barrier()
    pltpu.sync_copy(acc.at[pl.ds(s * _RPT, _RPT)],
                    out_hbm.at[c, pl.ds(s * _RPT, _RPT)])


def _proj_body(xa_ref, xb_ref, w_ref, o_ref):
    w = w_ref[...]
    a = jnp.dot(xa_ref[...], w, preferred_element_type=jnp.float32)
    b = jnp.dot(xb_ref[...], w, preferred_element_type=jnp.float32)
    o_ref[...] = jnp.concatenate([a, b], axis=1)


def _proj(x, w):
    return pl.pallas_call(
        _proj_body,
        grid=(_NB,),
        in_specs=[
            pl.BlockSpec((_BF, _D), lambda i: (i, 0)),
            pl.BlockSpec((_BF, _D), lambda i: (i + _NB, 0)),
            pl.BlockSpec((_D, _H), lambda i: (0, 0)),
        ],
        out_specs=pl.BlockSpec((_BF, 2 * _H), lambda i: (i, 0)),
        out_shape=jax.ShapeDtypeStruct((_NF, 2 * _H), jnp.float32),
    )(x, x, w)


def _mlp_body(has_next, p_ref, agg_ref, b1_ref, w2_ref, b2_ref, w1n_ref,
              bt_ref, bb_ref, *refs):
    if has_next:
        pn_ref, h_out, s_out, c_out = refs
    else:
        h_out, s_out, c_out = refs
    i = pl.program_id(0)

    @pl.when(i == 0)
    def _init():
        s_out[...] = jnp.zeros_like(s_out)
        c_out[...] = jnp.zeros_like(c_out)

    m = 2.0 * p_ref[...] + agg_ref[0] + agg_ref[1] + b1_ref[...]
    m = jnp.maximum(m, 0.0)
    w2 = w2_ref[...]
    b2 = b2_ref[...]
    h_top = jnp.maximum(
        jnp.dot(m[:, :_H], w2, preferred_element_type=jnp.float32) + b2, 0.0)
    h_bot = jnp.maximum(
        jnp.dot(m[:, _H:], w2, preferred_element_type=jnp.float32) + b2, 0.0)
    h_out[...] = jnp.concatenate([h_top, h_bot], axis=1)
    if has_next:
        w1n = w1n_ref[...]
        pn_ref[...] = jnp.concatenate(
            [jnp.dot(h_top, w1n, preferred_element_type=jnp.float32),
             jnp.dot(h_bot, w1n, preferred_element_type=jnp.float32)], axis=1)

    bt = bt_ref[...]
    bb = bb_ref[...]
    iota = lax.broadcasted_iota(jnp.int32, (_BF, _G), 1)
    oh_t = (bt == iota).astype(jnp.float32)
    oh_b = (bb == iota).astype(jnp.float32)
    dn = (((0,), (0,)), ((), ()))
    s_out[...] += (lax.dot_general(oh_t, h_top, dn, preferred_element_type=jnp.float32)
                   + lax.dot_general(oh_b, h_bot, dn, preferred_element_type=jnp.float32))
    ones = jnp.ones((_BF, 8), jnp.float32)
    c_out[...] += (lax.dot_general(oh_t, ones, dn, preferred_element_type=jnp.float32)
                   + lax.dot_general(oh_b, ones, dn, preferred_element_type=jnp.float32))

def _maxpool_body(h_ref, bt_ref, bb_ref, m_out):
    i = pl.program_id(0)

    @pl.when(i == 0)
    def _init():
        m_out[...] = jnp.full_like(m_out, -jnp.inf)

    h = h_ref[...]
    h_top = h[:, :_H]
    h_bot = h[:, _H:]
    bt = bt_ref[...]
    bb = bb_ref[...]
    rowid = lax.broadcasted_iota(jnp.int32, (_G, 1), 0)

    def g_top(g, carry):
        mg = jnp.max(jnp.where(bt == g, h_top, -jnp.inf), axis=0, keepdims=True)
        m_out[...] = jnp.maximum(m_out[...], jnp.where(rowid == g, mg, -jnp.inf))
        return carry

    def g_bot(g, carry):
        mg = jnp.max(jnp.where(bb == g, h_bot, -jnp.inf), axis=0, keepdims=True)
        m_out[...] = jnp.maximum(m_out[...], jnp.where(rowid == g, mg, -jnp.inf))
        return carry

    lax.fori_loop(bt[0, 0], bt[_BF - 1, 0] + 1, g_top, 0)
    lax.fori_loop(bb[0, 0], bb[_BF - 1, 0] + 1, g_bot, 0)


def _maxpool(h, bat):
    return pl.pallas_call(
        _maxpool_body,
        grid=(_NB,),
        in_specs=[
            pl.BlockSpec((_BF, 2 * _H), lambda i: (i, 0)),
            pl.BlockSpec((_BF, 1), lambda i: (i, 0)),
            pl.BlockSpec((_BF, 1), lambda i: (i + _NB, 0)),
        ],
        out_specs=pl.BlockSpec((_G, _H), lambda i: (0, 0)),
        out_shape=jax.ShapeDtypeStruct((_G, _H), jnp.float32),
    )(h, bat, bat)


def _mlp(p, agg, b1f, w2, b2, w1n, bat, has_next):
    in_specs = [
        pl.BlockSpec((_BF, 2 * _H), lambda i: (i, 0)),
        pl.BlockSpec((_NSC, _BF, 2 * _H), lambda i: (0, i, 0)),
        pl.BlockSpec((1, 2 * _H), lambda i: (0, 0)),
        pl.BlockSpec((_H, _H), lambda i: (0, 0)),
        pl.BlockSpec((1, _H), lambda i: (0, 0)),
        pl.BlockSpec((_H, _H), lambda i: (0, 0)),
        pl.BlockSpec((_BF, 1), lambda i: (i, 0)),
        pl.BlockSpec((_BF, 1), lambda i: (i + _NB, 0)),
    ]
    pool_specs = [pl.BlockSpec((_BF, 2 * _H), lambda i: (i, 0)),
                  pl.BlockSpec((_G, _H), lambda i: (0, 0)),
                  pl.BlockSpec((_G, 8), lambda i: (0, 0))]
    pool_shapes = [jax.ShapeDtypeStruct((_NF, 2 * _H), jnp.float32),
                   jax.ShapeDtypeStruct((_G, _H), jnp.float32),
                   jax.ShapeDtypeStruct((_G, 8), jnp.float32)]
    if has_next:
        out_specs = [pl.BlockSpec((_BF, 2 * _H), lambda i: (i, 0))] + pool_specs
        out_shape = [jax.ShapeDtypeStruct((_NF, 2 * _H), jnp.float32)] + pool_shapes
    else:
        out_specs = pool_specs
        out_shape = pool_shapes
    return pl.pallas_call(
        functools.partial(_mlp_body, has_next),
        grid=(_NB,),
        in_specs=in_specs,
        out_specs=out_specs,
        out_shape=out_shape,
    )(p, agg, b1f, w2, b2, w1n, bat, bat)


def _head_body(s1, s2, s3, m1, m2, m3, cnt_ref, fc1w_ref, fc1b_ref,
               fc2w_ref, fc2b_ref, out_ref):
    cnt = cnt_ref[:, 0:1]
    inv = 1.0 / jnp.maximum(cnt, 1.0)
    w = fc1w_ref[...]
    z = fc1b_ref[...]
    sums = [s1[...], s2[...], s3[...]]
    for k in range(3):
        mean_k = sums[k] * inv
        z = z + jnp.dot(mean_k, w[64 * k:64 * (k + 1)],
                        preferred_element_type=jnp.float32)
    maxs = [m1[...], m2[...], m3[...]]
    for k in range(3):
        mx_k = jnp.where(cnt > 0.0, maxs[k], 0.0)
        z = z + jnp.dot(mx_k, w[192 + 64 * k:192 + 64 * (k + 1)],
                        preferred_element_type=jnp.float32)
    for k in range(3):
        z = z + jnp.dot(sums[k], w[384 + 64 * k:384 + 64 * (k + 1)],
                        preferred_element_type=jnp.float32)
    z = jnp.maximum(z, 0.0)
    o = jnp.dot(z, fc2w_ref[...], preferred_element_type=jnp.float32) \
        + fc2b_ref[...]
    out_ref[...] = 1.0 / (1.0 + jnp.exp(-o))


def _head(pools, fc1_w, fc1_b, fc2_w, fc2_b):
    (s1, m1, c1), (s2, m2, _), (s3, m3, _) = pools
    gspec = lambda shape: pl.BlockSpec(shape, lambda: tuple(0 for _ in shape))
    return pl.pallas_call(
        _head_body,
        in_specs=[gspec((_G, _H))] * 6 + [
            gspec((_G, 8)), gspec((9 * _H, _H)), gspec((1, _H)),
            gspec((_H, _C)), gspec((1, _C)),
        ],
        out_specs=gspec((_G, _C)),
        out_shape=jax.ShapeDtypeStruct((_G, _C), jnp.float32),
    )(s1, s2, s3, m1, m2, m3, c1, fc1_w, fc1_b, fc2_w, fc2_b)


def _remap_fold(idx):
    return jnp.where(idx < _NF, 2 * idx, 2 * idx - (_N - 1))


def kernel(x, edge_index, batch,
           c0_w1, c0_b1, c0_w2, c0_b2,
           c1_w1, c1_b1, c1_w2, c1_b2,
           c2_w1, c2_b1, c2_w2, c2_b2,
           fc1_w, fc1_b, fc2_w, fc2_b):
    ed = _remap_fold(edge_index.astype(jnp.int32)).reshape(2, _NW, _NCH, _K)
    zeros = jnp.zeros((_NP, _H), jnp.float32)
    bat = batch.astype(jnp.int32).reshape(_N, 1)

    params = [(c0_b1, c0_w2, c0_b2), (c1_b1, c1_w2, c1_b2), (c2_b1, c2_w2, c2_b2)]
    next_w1 = [c1_w1, c2_w1, None]

    p = _proj(x, c0_w1)
    pools = []
    for l in range(3):
        agg = _sc_agg(p.reshape(_N, _H), ed, zeros)
        agg_f = agg.reshape(_NSC, _NP // 2, 2 * _H)
        b1, w2, b2 = params[l]
        b1f = jnp.concatenate([b1, b1]).reshape(1, 2 * _H)
        has_next = next_w1[l] is not None
        w1n = next_w1[l] if has_next else w2
        res = _mlp(p, agg_f, b1f, w2, b2.reshape(1, _H), w1n, bat, has_next)
        if has_next:
            p, h, s, c = res
        else:
            h, s, c = res
        pools.append((s, _maxpool(h, bat), c))

    return _head(pools, fc1_w, fc1_b.reshape(1, _H), fc2_w,
                 fc2_b.reshape(1, _C))
